# baseline (speedup 1.0000x reference)
import functools

import jax
import jax.numpy as jnp
from jax import lax
from jax.experimental import pallas as pl
from jax.experimental.pallas import tpu as pltpu
from jax.experimental.pallas import tpu_sc as plsc

_NC = 2
_NS = 16
_NW = _NC * _NS
_SUB = 8
_SW = 512
_CHUNK = 8
_SPLIT = 3


def _positional_lookup(table8, idx):
    n_sub, sw = table8.shape
    b = idx.shape[0]
    bpw = b // _NW
    n_chunks = bpw // _CHUNK
    csub = _CHUNK * _SUB
    stage_rows = 128
    n_stage_tiles = n_sub // stage_rows
    mesh = plsc.VectorSubcoreMesh(core_axis_name="c", subcore_axis_name="s")

    @functools.partial(
        pl.kernel,
        mesh=mesh,
        out_type=jax.ShapeDtypeStruct((b * _SUB, sw), jnp.float32),
        scratch_types=[
            pltpu.VMEM((bpw + 16,), jnp.int32),
            pltpu.VMEM((2, 4 * 16), jnp.int32),
            pltpu.VMEM((2, csub, sw), jnp.float32),
            pltpu.VMEM_SHARED((n_sub, sw), jnp.float32),
            pltpu.SemaphoreType.DMA,
            pltpu.SemaphoreType.DMA,
            pltpu.SemaphoreType.DMA,
            pltpu.SemaphoreType.DMA,
        ],
    )
    def k(idx_hbm, tab_hbm, out_hbm, idx_v, sidx_v, rows_v, tab_sp,
          g0, g1, w0, w1):
        gs = (g0, g1)
        ws = (w0, w1)
        sid = lax.axis_index("s")
        wid = sid * _NC + lax.axis_index("c")
        base = wid * bpw
        pltpu.sync_copy(idx_hbm.at[pl.ds(base, bpw)], idx_v.at[pl.ds(0, bpw)])

        @pl.when(sid < n_stage_tiles)
        def _():
            pltpu.sync_copy(
                tab_hbm.at[pl.ds(sid * stage_rows, stage_rows)],
                tab_sp.at[pl.ds(sid * stage_rows, stage_rows)],
            )

        plsc.subcore_barrier()

        lane = lax.iota(jnp.int32, 16)

        def fire_hbm(c, j):
            vec = idx_v[pl.ds(c * _CHUNK, 16)]
            for q in range(4):
                svals = vec[(lane >> 3) + 2 * q]
                sidx_v[j, pl.ds(q * 16, 16)] = (
                    (svals << 3) + (lane & 7)
                )
            pltpu.make_async_copy(
                tab_hbm.at[sidx_v.at[j]], rows_v.at[j], gs[j]
            ).start()

        def fire_spmem(c, j):
            vec = idx_v[pl.ds(c * _CHUNK, 16)]
            for r in range(_CHUNK):
                s = vec[r]
                pltpu.make_async_copy(
                    tab_sp.at[pl.ds(s * _SUB, _SUB)],
                    rows_v.at[j, pl.ds(r * _SUB, _SUB)],
                    gs[j],
                ).start()

        def fire_gathers(c, j):
            @pl.when(lax.rem(c, 8) < _SPLIT)
            def _():
                fire_spmem(c, j)

            @pl.when(lax.rem(c, 8) >= _SPLIT)
            def _():
                fire_hbm(c, j)

        def wait_gathers(j):
            pltpu.make_async_copy(
                tab_sp.at[pl.ds(0, csub)], rows_v.at[j], gs[j]
            ).wait()

        def write(c, j):
            return pltpu.make_async_copy(
                rows_v.at[j],
                out_hbm.at[pl.ds((base + c * _CHUNK) * _SUB, csub)],
                ws[j],
            )

        fire_gathers(0, 0)

        def body(i, carry):
            for j in range(2):
                c = 2 * i + j
                wait_gathers(j)
                write(c, j).start()
                cn = c + 1

                @pl.when(cn < n_chunks)
                def _():
                    @pl.when(cn >= 2)
                    def _():
                        write(cn - 2, 1 - j).wait()

                    fire_gathers(cn, 1 - j)

            return carry

        lax.fori_loop(0, n_chunks // 2, body, 0)
        write(n_chunks - 2, 0).wait()
        write(n_chunks - 1, 1).wait()

    return k(idx, table8)


def kernel(x, weights):
    n_rows = weights.shape[0]
    d = weights.shape[1] * weights.shape[2]
    table = weights.reshape(n_rows, d)
    pad = (-n_rows) % 16
    if pad:
        table = jnp.pad(table, ((0, pad), (0, 0)))
    table8 = table.reshape(-1, _SW)
    out = _positional_lookup(table8, x)
    return out.reshape(x.shape[0], d)


# speedup vs baseline: 2.2708x; 2.2708x over previous
import functools

import jax
import jax.numpy as jnp
from jax import lax
from jax.experimental import pallas as pl
from jax.experimental.pallas import tpu as pltpu
from jax.experimental.pallas import tpu_sc as plsc

_NC = 2
_NS = 16
_NW = _NC * _NS


def _positional_lookup(table, idx, *, chunk):
    n_rows, d = table.shape
    b = idx.shape[0]
    bpw = b // _NW
    n_chunks = bpw // chunk
    mesh = plsc.VectorSubcoreMesh(core_axis_name="c", subcore_axis_name="s")

    @functools.partial(
        pl.kernel,
        mesh=mesh,
        out_type=jax.ShapeDtypeStruct((b, d), jnp.float32),
        scratch_types=[
            pltpu.VMEM((bpw,), jnp.int32),
            pltpu.VMEM((2, chunk, d), jnp.float32),
            pltpu.SemaphoreType.DMA,
            pltpu.SemaphoreType.DMA,
            pltpu.SemaphoreType.DMA,
            pltpu.SemaphoreType.DMA,
        ],
    )
    def k(idx_hbm, tab_hbm, out_hbm, idx_v, rows_v, g0, g1, w0, w1):
        gs = (g0, g1)
        ws = (w0, w1)
        wid = lax.axis_index("s") * _NC + lax.axis_index("c")
        base = wid * bpw
        pltpu.sync_copy(idx_hbm.at[pl.ds(base, bpw)], idx_v)

        def gather(c, j):
            return pltpu.make_async_copy(
                tab_hbm.at[idx_v.at[pl.ds(c * chunk, chunk)]],
                rows_v.at[j],
                gs[j],
            )

        def write(c, j):
            return pltpu.make_async_copy(
                rows_v.at[j],
                out_hbm.at[pl.ds(base + c * chunk, chunk)],
                ws[j],
            )

        gather(0, 0).start()

        def body(i, carry):
            for j in range(2):
                c = 2 * i + j
                gather(c, j).wait()
                write(c, j).start()
                cn = c + 1

                @pl.when(cn < n_chunks)
                def _():
                    @pl.when(cn >= 2)
                    def _():
                        write(cn - 2, 1 - j).wait()

                    gather(cn, 1 - j).start()

            return carry

        lax.fori_loop(0, n_chunks // 2, body, 0)
        write(n_chunks - 2, 0).wait()
        write(n_chunks - 1, 1).wait()

    return k(idx, table)


def kernel(x, weights):
    n_rows = weights.shape[0]
    d = weights.shape[1] * weights.shape[2]
    table = weights.reshape(n_rows, d)
    out = _positional_lookup(table, x, chunk=8)
    return out
